# baseline (speedup 1.0000x reference)
import functools

import jax
import jax.numpy as jnp
from jax import lax
from jax.experimental import pallas as pl
from jax.experimental.pallas import tpu as pltpu
from jax.experimental.pallas import tpu_sc as plsc

D = 32
NC = 2
NS = 16
NW = NC * NS


def _make_transpose(vocab: int):
    n_full = vocab // 128
    rem = vocab - n_full * 128
    per_w = n_full // NW
    extra = n_full - per_w * NW

    mesh = plsc.VectorSubcoreMesh(core_axis_name="c", subcore_axis_name="s")

    @functools.partial(
        pl.kernel,
        mesh=mesh,
        compiler_params=pltpu.CompilerParams(use_tc_tiling_on_sc=True,
                                             needs_layout_passes=False),
        out_type=jax.ShapeDtypeStruct((vocab // 4, 4 * D), jnp.float32),
        scratch_types=[
            pltpu.VMEM((D, 128), jnp.float32),
            pltpu.VMEM((D, 128), jnp.float32),
            pltpu.VMEM((32, 128), jnp.float32),
            pltpu.VMEM((32, 128), jnp.float32),
            pltpu.VMEM((D, 64), jnp.float32),
            pltpu.VMEM((16, 128), jnp.float32),
            pltpu.SemaphoreType.DMA,
            pltpu.SemaphoreType.DMA,
            pltpu.SemaphoreType.DMA,
            pltpu.SemaphoreType.DMA,
        ],
    )
    def tp_kernel(tt_hbm, tbl4_hbm, src0, src1, dst0, dst1, src_r, dst_r,
                  isem0, isem1, osem0, osem1):
        wid = lax.axis_index("s") * NC + lax.axis_index("c")
        lo = wid * per_w + jnp.minimum(wid, extra)
        hi = lo + per_w + jnp.where(wid < extra, 1, 0)
        srcs = (src0, src1)
        dsts = (dst0, dst1)
        isems = (isem0, isem1)
        osems = (osem0, osem1)
        iota = lax.iota(jnp.int32, 16)

        def start_in(i, q):
            pltpu.async_copy(tt_hbm.at[:, pl.ds(i * 128, 128)], srcs[q],
                             isems[q])

        def wait_in(i, q):
            pltpu.make_async_copy(tt_hbm.at[:, pl.ds(i * 128, 128)], srcs[q],
                                  isems[q]).wait()

        def start_out(i, q):
            pltpu.async_copy(dsts[q], tbl4_hbm.at[pl.ds(i * 32, 32)],
                             osems[q])

        def wait_out(i, q):
            pltpu.make_async_copy(dsts[q], tbl4_hbm.at[pl.ds(i * 32, 32)],
                                  osems[q]).wait()

        def transpose_block(q):
            src, dst = srcs[q], dsts[q]

            @pl.loop(0, 128 // 16)
            def _(vb):
                v = vb * 16 + iota
                row = lax.shift_right_logical(v, 2)
                colb = lax.bitwise_and(v, 3) * D
                for c in range(D):
                    vals = src[c, pl.ds(vb * 16, 16)]
                    plsc.store_scatter(dst, [row, colb + c], vals)

        @pl.when(lo < hi)
        def _():
            start_in(lo, 0)

        @pl.when(lo + 1 < hi)
        def _():
            start_in(lo + 1, 1)

        n_iter = per_w // 2 + 1

        @pl.loop(0, n_iter)
        def _(p):
            for q in (0, 1):
                i = lo + 2 * p + q

                @pl.when(i < hi)
                def _():
                    wait_in(i, q)

                    @pl.when(p > 0)
                    def _():
                        wait_out(i - 2, q)

                    transpose_block(q)
                    start_out(i, q)

                    @pl.when(i + 2 < hi)
                    def _():
                        start_in(i + 2, q)

        n_blk = hi - lo
        for q in (0, 1):
            last = hi - 1 - ((n_blk - 1 - q) % 2)

            @pl.when((n_blk > q) & (last >= lo))
            def _():
                wait_out(last, q)

        if rem:
            @pl.when(wid == NW - 1)
            def _():
                for c in range(D):
                    pltpu.sync_copy(tt_hbm.at[c, pl.ds(n_full * 128, rem)],
                                    src_r.at[c])
                @pl.loop(0, D)
                def _(c):
                    for v0 in range(0, rem, 16):
                        vals = src_r[c, pl.ds(v0, 16)]
                        pos = (v0 + iota) * D + c
                        plsc.store_scatter(
                            dst_r, [lax.shift_right_logical(pos, 7),
                                    lax.bitwise_and(pos, 127)], vals)
                pltpu.sync_copy(
                    dst_r, tbl4_hbm.at[pl.ds(n_full * 32, rem * D // 128)])

    return tp_kernel


def _make_gather(batch: int, hist: int, vocab: int):
    assert batch % NW == 0
    strip = batch // NW
    half = strip // 2
    n_units = 2 * hist

    mesh = plsc.VectorSubcoreMesh(core_axis_name="c", subcore_axis_name="s")

    @functools.partial(
        pl.kernel,
        mesh=mesh,
        compiler_params=pltpu.CompilerParams(use_tc_tiling_on_sc=True,
                                             needs_layout_passes=False),
        out_type=jax.ShapeDtypeStruct((hist, D, batch), jnp.float32),
        scratch_types=[
            pltpu.VMEM((hist, strip), jnp.int32),
            pltpu.VMEM((half,), jnp.int32),
            pltpu.VMEM((half,), jnp.int32),
            pltpu.VMEM((half,), jnp.int32),
            pltpu.VMEM((half,), jnp.int32),
            pltpu.VMEM((half, 4 * D), jnp.float32),
            pltpu.VMEM((half, 4 * D), jnp.float32),
            pltpu.VMEM((D, half), jnp.float32),
            pltpu.VMEM((D, half), jnp.float32),
            pltpu.SemaphoreType.DMA,
            pltpu.SemaphoreType.DMA,
            pltpu.SemaphoreType.DMA,
            pltpu.SemaphoreType.DMA,
        ],
    )
    def gather_kernel(xt_hbm, tbl4_hbm, out_hbm,
                      raw_all, idx0, idx1, off0, off1,
                      rows0, rows1, rt0, rt1,
                      gsem0, gsem1, osem0, osem1):
        wid = lax.axis_index("s") * NC + lax.axis_index("c")
        b0 = wid * strip
        idxs = (idx0, idx1)
        offs = (off0, off1)
        rows = (rows0, rows1)
        rts = (rt0, rt1)
        gsems = (gsem0, gsem1)
        osems = (osem0, osem1)
        iota = lax.iota(jnp.int32, 16)

        pltpu.sync_copy(xt_hbm.at[:, pl.ds(b0, strip)], raw_all)

        def prep(u, q):
            h = lax.div(u, 2)
            sub = lax.rem(u, 2)
            for k in range(0, half, 16):
                v = raw_all[h, pl.ds(sub * half + k, 16)]
                idxs[q][pl.ds(k, 16)] = lax.shift_right_logical(v, 2)
                offs[q][pl.ds(k, 16)] = lax.bitwise_and(v, 3) * D

        def start_gather(q):
            pltpu.async_copy(tbl4_hbm.at[idxs[q]], rows[q], gsems[q])

        def wait_gather(q):
            pltpu.make_async_copy(tbl4_hbm.at[idxs[q]], rows[q],
                                  gsems[q]).wait()

        def oslice(u):
            h = lax.div(u, 2)
            sub = lax.rem(u, 2)
            return out_hbm.at[h, :, pl.ds(b0 + sub * half, half)]

        def start_out(u, q):
            pltpu.async_copy(rts[q], oslice(u), osems[q])

        def wait_out(u, q):
            pltpu.make_async_copy(rts[q], oslice(u), osems[q]).wait()

        def extract(q):
            src, off, dst = rows[q], offs[q], rts[q]

            @pl.loop(0, half // 16)
            def _(kb):
                k = kb * 16
                o = off[pl.ds(k, 16)]
                row = k + iota
                for c in range(D):
                    vals = plsc.load_gather(src, [row, o + c])
                    dst[c, pl.ds(k, 16)] = vals

        for q in (0, 1):
            prep(q, q)
            start_gather(q)

        @pl.loop(0, n_units // 2)
        def _(p):
            for q in (0, 1):
                u = 2 * p + q

                wait_gather(q)

                @pl.when(u >= 2)
                def _():
                    wait_out(u - 2, q)

                extract(q)
                start_out(u, q)

                @pl.when(u + 2 < n_units)
                def _():
                    prep(u + 2, q)
                    start_gather(q)

        for q in (0, 1):
            wait_out(n_units - 2 + q, q)

    return gather_kernel


@jax.jit
def kernel(x, table):
    batch, hist = x.shape
    vocab = table.shape[0]
    tp = _make_transpose(vocab)
    gather = _make_gather(batch, hist, vocab)
    tbl4 = tp(table.T)
    out_t = gather(x.T.astype(jnp.int32), tbl4)
    return jnp.transpose(out_t, (2, 0, 1))


# speedup vs baseline: 1.2486x; 1.2486x over previous
import functools

import jax
import jax.numpy as jnp
from jax import lax
from jax.experimental import pallas as pl
from jax.experimental.pallas import tpu as pltpu
from jax.experimental.pallas import tpu_sc as plsc

EMBED_DIM = 32
NUM_CORES = 2
NUM_SUBCORES = 16
NUM_WORKERS = NUM_CORES * NUM_SUBCORES


def _make_gather(batch: int, hist: int, vocab: int, chunk_rows: int):
    assert batch % NUM_WORKERS == 0
    rows_per_w = batch // NUM_WORKERS
    assert rows_per_w % chunk_rows == 0
    n_chunks = rows_per_w // chunk_rows

    mesh = plsc.VectorSubcoreMesh(core_axis_name="c", subcore_axis_name="s")

    @functools.partial(
        pl.kernel,
        mesh=mesh,
        compiler_params=pltpu.CompilerParams(use_tc_tiling_on_sc=False),
        out_type=jax.ShapeDtypeStruct((batch, hist, EMBED_DIM), jnp.float32),
        scratch_types=[
            pltpu.VMEM((rows_per_w, hist), jnp.int32),
            pltpu.VMEM((chunk_rows, hist, EMBED_DIM), jnp.float32),
            pltpu.VMEM((chunk_rows, hist, EMBED_DIM), jnp.float32),
            pltpu.SemaphoreType.DMA,
            pltpu.SemaphoreType.DMA,
        ],
    )
    def gather_kernel(x_hbm, table_hbm, out_hbm, idx_v, rows0, rows1, sem0, sem1):
        wid = lax.axis_index("s") * NUM_CORES + lax.axis_index("c")
        base = wid * rows_per_w
        pltpu.sync_copy(x_hbm.at[pl.ds(base, rows_per_w)], idx_v)

        rows = (rows0, rows1)
        sems = (sem0, sem1)

        def start(i, buf):
            for j in range(chunk_rows):
                pltpu.async_copy(
                    table_hbm.at[idx_v.at[i * chunk_rows + j]],
                    rows[buf].at[j],
                    sems[buf],
                )

        def drain(i, buf):
            for j in range(chunk_rows):
                pltpu.make_async_copy(
                    table_hbm.at[idx_v.at[i * chunk_rows + j]],
                    rows[buf].at[j],
                    sems[buf],
                ).wait()

        start(0, 0)

        @pl.loop(0, n_chunks // 2)
        def _body(p):
            i = p * 2
            start(i + 1, 1)
            drain(i, 0)
            pltpu.sync_copy(rows[0],
                            out_hbm.at[pl.ds(base + i * chunk_rows, chunk_rows)])
            @pl.when(i + 2 < n_chunks)
            def _():
                start(i + 2, 0)
            drain(i + 1, 1)
            pltpu.sync_copy(rows[1],
                            out_hbm.at[pl.ds(base + (i + 1) * chunk_rows,
                                             chunk_rows)])

    return gather_kernel


@jax.jit
def kernel(x, table):
    batch, hist = x.shape
    gather = _make_gather(batch, hist, table.shape[0], chunk_rows=8)
    return gather(x.astype(jnp.int32), table)
